# baseline (speedup 1.0000x reference)
import functools
import math

import jax
import jax.numpy as jnp
from jax import lax
from jax.experimental import pallas as pl
from jax.experimental.pallas import tpu as pltpu
from jax.experimental.pallas import tpu_sc as plsc

N = 10000
F = 128
E = 320000
NC, NS = 2, 16
NW = NC * NS
CH = 128
C0_CH = 62
C1_CH = 98
EPAD = NS * (C0_CH + C1_CH) * CH
NPAD = 10240
RPT = NPAD // NS
RB = 1000
TWO_PI = 2.0 * math.pi
INV_SQRT2 = 1.0 / math.sqrt(2.0)


def _embed_body(x_ref, b_ref, wp_ref, bp_ref, out_ref):
    xp = TWO_PI * jnp.dot(x_ref[...], b_ref[...],
                          preferred_element_type=jnp.float32)
    dn = (((1,), (1,)), ((), ()))
    out_ref[...] = (
        lax.dot_general(jnp.cos(xp), wp_ref[...][:, :F], dn,
                        preferred_element_type=jnp.float32)
        + lax.dot_general(jnp.sin(xp), wp_ref[...][:, F:], dn,
                          preferred_element_type=jnp.float32)
        + bp_ref[...])


def _embed(x, B, W_proj, b_proj):
    return pl.pallas_call(
        _embed_body,
        grid=(N // RB,),
        in_specs=[
            pl.BlockSpec((RB, F), lambda i: (i, 0)),
            pl.BlockSpec((F, F), lambda i: (0, 0)),
            pl.BlockSpec((F, 2 * F), lambda i: (0, 0)),
            pl.BlockSpec((1, F), lambda i: (0, 0)),
        ],
        out_specs=pl.BlockSpec((RB, F), lambda i: (i, 0)),
        out_shape=jax.ShapeDtypeStruct((N, F), jnp.float32),
    )(x, B, W_proj, b_proj.reshape(1, F))


def _sc_body(src_h, dst_h, xe_h, z2_h, z1_h, sums_h, cnts_h,
             acc_sh, cnt_sh, sidx, didx, rows, ones, sem):
    c = lax.axis_index("c")
    s = lax.axis_index("s")
    for j in range(CH // 16):
        ones[pl.ds(j * 16, 16)] = jnp.ones((16,), jnp.float32)
    pltpu.sync_copy(z2_h, acc_sh.at[pl.ds(s * RPT, RPT)])
    pltpu.sync_copy(z1_h, cnt_sh.at[pl.ds(s * RPT, RPT)])
    plsc.subcore_barrier()
    base = jnp.where(c == 0, s * C0_CH, NS * C0_CH + s * C1_CH) * CH
    nch = jnp.where(c == 0, C0_CH, C1_CH)

    def chunk(k, carry):
        off = base + k * CH
        pltpu.sync_copy(src_h.at[pl.ds(off, CH)], sidx)
        pltpu.sync_copy(dst_h.at[pl.ds(off, CH)], didx)
        pltpu.async_copy(xe_h.at[sidx], rows, sem).wait()
        pltpu.sync_copy(rows, acc_sh.at[didx], add=True)
        pltpu.sync_copy(ones, cnt_sh.at[didx], add=True)
        return carry

    lax.fori_loop(0, nch, chunk, 0)
    plsc.subcore_barrier()
    pltpu.sync_copy(acc_sh.at[pl.ds(s * RPT, RPT)],
                    sums_h.at[c, pl.ds(s * RPT, RPT)])
    pltpu.sync_copy(cnt_sh.at[pl.ds(s * RPT, RPT)],
                    cnts_h.at[c, pl.ds(s * RPT, RPT)])


def _aggregate(src_p, dst_p, x_embed):
    z2 = jnp.zeros((RPT, F), jnp.float32)
    z1 = jnp.zeros((RPT,), jnp.float32)
    run = pl.kernel(
        _sc_body,
        out_type=[
            jax.ShapeDtypeStruct((NC, NPAD, F), jnp.float32),
            jax.ShapeDtypeStruct((NC, NPAD), jnp.float32),
        ],
        mesh=plsc.VectorSubcoreMesh(core_axis_name="c", subcore_axis_name="s"),
        scratch_types=[
            pltpu.VMEM_SHARED((NPAD, F), jnp.float32),
            pltpu.VMEM_SHARED((NPAD,), jnp.float32),
            pltpu.VMEM((CH,), jnp.int32),
            pltpu.VMEM((CH,), jnp.int32),
            pltpu.VMEM((CH, F), jnp.float32),
            pltpu.VMEM((CH,), jnp.float32),
            pltpu.SemaphoreType.DMA,
        ],
    )
    return run(src_p, dst_p, x_embed, z2, z1)


def _fuse_body(xe_ref, p_ref, cnt_ref, wf_ref, bf_ref,
               fused_ref, s_ref, q_ref):
    i = pl.program_id(0)
    summed = p_ref[0] + p_ref[1]
    cnt = cnt_ref[0] + cnt_ref[1]
    aggs = jnp.where(cnt > 0.0, summed / jnp.maximum(cnt, 1.0), 0.0)
    dn = (((1,), (1,)), ((), ()))
    fused = (
        lax.dot_general(xe_ref[...], wf_ref[...][:, :F], dn,
                        preferred_element_type=jnp.float32)
        + lax.dot_general(aggs, wf_ref[...][:, F:], dn,
                          preferred_element_type=jnp.float32)
        + bf_ref[...])
    fused_ref[...] = fused

    @pl.when(i == 0)
    def _():
        s_ref[...] = jnp.zeros_like(s_ref)
        q_ref[...] = jnp.zeros_like(q_ref)

    s_ref[...] += jnp.sum(fused, axis=0, keepdims=True)
    q_ref[...] += jnp.sum(fused * fused, axis=0, keepdims=True)


def _fuse(x_embed, sums, cnts, W_fus, b_fus):
    return pl.pallas_call(
        _fuse_body,
        grid=(N // RB,),
        in_specs=[
            pl.BlockSpec((RB, F), lambda i: (i, 0)),
            pl.BlockSpec((NC, RB, F), lambda i: (0, i, 0)),
            pl.BlockSpec((NC, RB, 1), lambda i: (0, i, 0)),
            pl.BlockSpec((F, 2 * F), lambda i: (0, 0)),
            pl.BlockSpec((1, F), lambda i: (0, 0)),
        ],
        out_specs=[
            pl.BlockSpec((RB, F), lambda i: (i, 0)),
            pl.BlockSpec((1, F), lambda i: (0, 0)),
            pl.BlockSpec((1, F), lambda i: (0, 0)),
        ],
        out_shape=[
            jax.ShapeDtypeStruct((N, F), jnp.float32),
            jax.ShapeDtypeStruct((1, F), jnp.float32),
            jax.ShapeDtypeStruct((1, F), jnp.float32),
        ],
    )(x_embed, sums, cnts.reshape(NC, NPAD, 1), W_fus, b_fus.reshape(1, F))


def _bn_body(fused_ref, s_ref, q_ref, g_ref, bt_ref, out_ref):
    mean = s_ref[...] * (1.0 / N)
    var = q_ref[...] * (1.0 / N) - mean * mean
    inv = lax.rsqrt(var + 1e-5)
    xh = (fused_ref[...] - mean) * inv * g_ref[...] + bt_ref[...]
    out_ref[...] = 0.5 * xh * (1.0 + lax.erf(xh * INV_SQRT2))


def _bn_gelu(fused, ssum, sq, gamma, beta):
    return pl.pallas_call(
        _bn_body,
        grid=(N // RB,),
        in_specs=[
            pl.BlockSpec((RB, F), lambda i: (i, 0)),
            pl.BlockSpec((1, F), lambda i: (0, 0)),
            pl.BlockSpec((1, F), lambda i: (0, 0)),
            pl.BlockSpec((1, F), lambda i: (0, 0)),
            pl.BlockSpec((1, F), lambda i: (0, 0)),
        ],
        out_specs=pl.BlockSpec((RB, F), lambda i: (i, 0)),
        out_shape=jax.ShapeDtypeStruct((N, F), jnp.float32),
    )(fused, ssum, sq, gamma.reshape(1, F), beta.reshape(1, F))


def kernel(x, edge_index, B, W_proj, b_proj, W_fus, b_fus, gamma, beta):
    x_embed = _embed(x, B, W_proj, b_proj)
    src_p = jnp.concatenate(
        [edge_index[0], jnp.zeros((EPAD - E,), jnp.int32)])
    dst_p = jnp.concatenate(
        [edge_index[1], jnp.full((EPAD - E,), N, jnp.int32)])
    sums, cnts = _aggregate(src_p, dst_p, x_embed)
    fused, ssum, sq = _fuse(x_embed, sums, cnts, W_fus, b_fus)
    return _bn_gelu(fused, ssum, sq, gamma, beta)


# speedup vs baseline: 1.0719x; 1.0719x over previous
import functools
import math

import jax
import jax.numpy as jnp
from jax import lax
from jax.experimental import pallas as pl
from jax.experimental.pallas import tpu as pltpu
from jax.experimental.pallas import tpu_sc as plsc

N = 10000
F = 128
E = 320000
NC, NS = 2, 16
NW = NC * NS
CH = 128
NCH = 80
EPT = NCH * CH
EPAD = EPT * NW
NPAD = 10240
RPT = NPAD // NS
RB = 1000
TWO_PI = 2.0 * math.pi
INV_SQRT2 = 1.0 / math.sqrt(2.0)


def _embed_body(x_ref, b_ref, wp_ref, bp_ref, out_ref):
    xp = TWO_PI * jnp.dot(x_ref[...], b_ref[...],
                          preferred_element_type=jnp.float32)
    dn = (((1,), (1,)), ((), ()))
    out_ref[...] = (
        lax.dot_general(jnp.cos(xp), wp_ref[...][:, :F], dn,
                        preferred_element_type=jnp.float32)
        + lax.dot_general(jnp.sin(xp), wp_ref[...][:, F:], dn,
                          preferred_element_type=jnp.float32)
        + bp_ref[...])


def _embed(x, B, W_proj, b_proj):
    return pl.pallas_call(
        _embed_body,
        grid=(N // RB,),
        in_specs=[
            pl.BlockSpec((RB, F), lambda i: (i, 0)),
            pl.BlockSpec((F, F), lambda i: (0, 0)),
            pl.BlockSpec((F, 2 * F), lambda i: (0, 0)),
            pl.BlockSpec((1, F), lambda i: (0, 0)),
        ],
        out_specs=pl.BlockSpec((RB, F), lambda i: (i, 0)),
        out_shape=jax.ShapeDtypeStruct((N, F), jnp.float32),
    )(x, B, W_proj, b_proj.reshape(1, F))


def _sc_body(src_h, dst_h, xe_h, z2_h, z1_h, sums_h, cnts_h,
             acc_sh, cnt_sh, sidx, didx, rows, ones, sem):
    c = lax.axis_index("c")
    s = lax.axis_index("s")
    for j in range(CH // 16):
        ones[pl.ds(j * 16, 16)] = jnp.ones((16,), jnp.float32)
    pltpu.sync_copy(z2_h, acc_sh.at[pl.ds(s * RPT, RPT)])
    pltpu.sync_copy(z1_h, cnt_sh.at[pl.ds(s * RPT, RPT)])
    plsc.subcore_barrier()
    wid = c * NS + s
    base = wid * EPT
    half = CH // 2

    def chunk(k, carry):
        off = base + k * CH
        pltpu.sync_copy(src_h.at[pl.ds(off, CH)], sidx)
        pltpu.sync_copy(dst_h.at[pl.ds(off, CH)], didx)
        d0 = pltpu.async_copy(xe_h.at[sidx.at[pl.ds(0, half)]],
                              rows.at[pl.ds(0, half)], sem)
        d1 = pltpu.async_copy(xe_h.at[sidx.at[pl.ds(half, half)]],
                              rows.at[pl.ds(half, half)], sem)
        d0.wait()
        d1.wait()
        pltpu.sync_copy(rows, acc_sh.at[didx], add=True)
        pltpu.sync_copy(ones, cnt_sh.at[didx], add=True)
        return carry

    lax.fori_loop(0, NCH, chunk, 0)
    plsc.subcore_barrier()
    pltpu.sync_copy(acc_sh.at[pl.ds(s * RPT, RPT)],
                    sums_h.at[c, pl.ds(s * RPT, RPT)])
    pltpu.sync_copy(cnt_sh.at[pl.ds(s * RPT, RPT)],
                    cnts_h.at[c, pl.ds(s * RPT, RPT)])


def _aggregate(src_p, dst_p, x_embed):
    z2 = jnp.zeros((RPT, F), jnp.float32)
    z1 = jnp.zeros((RPT,), jnp.float32)
    run = pl.kernel(
        _sc_body,
        out_type=[
            jax.ShapeDtypeStruct((NC, NPAD, F), jnp.float32),
            jax.ShapeDtypeStruct((NC, NPAD), jnp.float32),
        ],
        mesh=plsc.VectorSubcoreMesh(core_axis_name="c", subcore_axis_name="s"),
        scratch_types=[
            pltpu.VMEM_SHARED((NPAD, F), jnp.float32),
            pltpu.VMEM_SHARED((NPAD,), jnp.float32),
            pltpu.VMEM((CH,), jnp.int32),
            pltpu.VMEM((CH,), jnp.int32),
            pltpu.VMEM((CH, F), jnp.float32),
            pltpu.VMEM((CH,), jnp.float32),
            pltpu.SemaphoreType.DMA,
        ],
    )
    return run(src_p, dst_p, x_embed, z2, z1)


def _fuse_body(xe_ref, p_ref, cnt_ref, wf_ref, bf_ref,
               fused_ref, s_ref, q_ref):
    i = pl.program_id(0)
    summed = p_ref[0] + p_ref[1]
    cnt = cnt_ref[0] + cnt_ref[1]
    aggs = jnp.where(cnt > 0.0, summed / jnp.maximum(cnt, 1.0), 0.0)
    dn = (((1,), (1,)), ((), ()))
    fused = (
        lax.dot_general(xe_ref[...], wf_ref[...][:, :F], dn,
                        preferred_element_type=jnp.float32)
        + lax.dot_general(aggs, wf_ref[...][:, F:], dn,
                          preferred_element_type=jnp.float32)
        + bf_ref[...])
    fused_ref[...] = fused

    @pl.when(i == 0)
    def _():
        s_ref[...] = jnp.zeros_like(s_ref)
        q_ref[...] = jnp.zeros_like(q_ref)

    s_ref[...] += jnp.sum(fused, axis=0, keepdims=True)
    q_ref[...] += jnp.sum(fused * fused, axis=0, keepdims=True)


def _fuse(x_embed, sums, cnts, W_fus, b_fus):
    return pl.pallas_call(
        _fuse_body,
        grid=(N // RB,),
        in_specs=[
            pl.BlockSpec((RB, F), lambda i: (i, 0)),
            pl.BlockSpec((NC, RB, F), lambda i: (0, i, 0)),
            pl.BlockSpec((NC, RB, 1), lambda i: (0, i, 0)),
            pl.BlockSpec((F, 2 * F), lambda i: (0, 0)),
            pl.BlockSpec((1, F), lambda i: (0, 0)),
        ],
        out_specs=[
            pl.BlockSpec((RB, F), lambda i: (i, 0)),
            pl.BlockSpec((1, F), lambda i: (0, 0)),
            pl.BlockSpec((1, F), lambda i: (0, 0)),
        ],
        out_shape=[
            jax.ShapeDtypeStruct((N, F), jnp.float32),
            jax.ShapeDtypeStruct((1, F), jnp.float32),
            jax.ShapeDtypeStruct((1, F), jnp.float32),
        ],
    )(x_embed, sums, cnts.reshape(NC, NPAD, 1), W_fus, b_fus.reshape(1, F))


def _bn_body(fused_ref, s_ref, q_ref, g_ref, bt_ref, out_ref):
    mean = s_ref[...] * (1.0 / N)
    var = q_ref[...] * (1.0 / N) - mean * mean
    inv = lax.rsqrt(var + 1e-5)
    xh = (fused_ref[...] - mean) * inv * g_ref[...] + bt_ref[...]
    out_ref[...] = 0.5 * xh * (1.0 + lax.erf(xh * INV_SQRT2))


def _bn_gelu(fused, ssum, sq, gamma, beta):
    return pl.pallas_call(
        _bn_body,
        grid=(N // RB,),
        in_specs=[
            pl.BlockSpec((RB, F), lambda i: (i, 0)),
            pl.BlockSpec((1, F), lambda i: (0, 0)),
            pl.BlockSpec((1, F), lambda i: (0, 0)),
            pl.BlockSpec((1, F), lambda i: (0, 0)),
            pl.BlockSpec((1, F), lambda i: (0, 0)),
        ],
        out_specs=pl.BlockSpec((RB, F), lambda i: (i, 0)),
        out_shape=jax.ShapeDtypeStruct((N, F), jnp.float32),
    )(fused, ssum, sq, gamma.reshape(1, F), beta.reshape(1, F))


def kernel(x, edge_index, B, W_proj, b_proj, W_fus, b_fus, gamma, beta):
    x_embed = _embed(x, B, W_proj, b_proj)
    src_p = jnp.concatenate(
        [edge_index[0], jnp.zeros((EPAD - E,), jnp.int32)])
    dst_p = jnp.concatenate(
        [edge_index[1], jnp.full((EPAD - E,), N, jnp.int32)])
    sums, cnts = _aggregate(src_p, dst_p, x_embed)
    fused, ssum, sq = _fuse(x_embed, sums, cnts, W_fus, b_fus)
    return _bn_gelu(fused, ssum, sq, gamma, beta)


# speedup vs baseline: 1.3775x; 1.2851x over previous
import functools
import math

import jax
import jax.numpy as jnp
from jax import lax
from jax.experimental import pallas as pl
from jax.experimental.pallas import tpu as pltpu
from jax.experimental.pallas import tpu_sc as plsc

N = 10000
F = 128
E = 320000
NC, NS = 2, 16
NW = NC * NS
CH = 128
EPT = 10112
EPAD = EPT * NW
NPAD = 10240
RPT = NPAD // NS
RB = 1000
TWO_PI = 2.0 * math.pi
INV_SQRT2 = 1.0 / math.sqrt(2.0)


def _embed_body(x_ref, b_ref, wp_ref, bp_ref, out_ref):
    xp = TWO_PI * jnp.dot(x_ref[...], b_ref[...],
                          preferred_element_type=jnp.float32)
    dn = (((1,), (1,)), ((), ()))
    out_ref[...] = (
        lax.dot_general(jnp.cos(xp), wp_ref[...][:, :F], dn,
                        preferred_element_type=jnp.float32)
        + lax.dot_general(jnp.sin(xp), wp_ref[...][:, F:], dn,
                          preferred_element_type=jnp.float32)
        + bp_ref[...])


def _embed(x, B, W_proj, b_proj):
    return pl.pallas_call(
        _embed_body,
        grid=(N // RB,),
        in_specs=[
            pl.BlockSpec((RB, F), lambda i: (i, 0)),
            pl.BlockSpec((F, F), lambda i: (0, 0)),
            pl.BlockSpec((F, 2 * F), lambda i: (0, 0)),
            pl.BlockSpec((1, F), lambda i: (0, 0)),
        ],
        out_specs=pl.BlockSpec((RB, F), lambda i: (i, 0)),
        out_shape=jax.ShapeDtypeStruct((N, F), jnp.float32),
    )(x, B, W_proj, b_proj.reshape(1, F))


def _sc_body(src_h, dst_h, xe_h, z2_h, z1_h, sums_h, cnts_h,
             acc_sh, cnt_sh, sidx, didx, rows, ones, sem):
    c = lax.axis_index("c")
    s = lax.axis_index("s")
    wid = c * NS + s
    for j in range(CH // 16):
        ones[pl.ds(j * 16, 16)] = jnp.ones((16,), jnp.float32)
    pltpu.sync_copy(z2_h, acc_sh.at[pl.ds(s * RPT, RPT)])
    pltpu.sync_copy(z1_h, cnt_sh.at[pl.ds(s * RPT, RPT)])
    plsc.subcore_barrier()
    base = wid * EPT

    def chunk(k, carry):
        off = base + k * CH
        pltpu.sync_copy(src_h.at[pl.ds(off, CH)], sidx)
        pltpu.sync_copy(dst_h.at[pl.ds(off, CH)], didx)
        pltpu.async_copy(xe_h.at[sidx], rows, sem).wait()
        pltpu.sync_copy(rows, acc_sh.at[didx], add=True)
        pltpu.sync_copy(ones, cnt_sh.at[didx], add=True)
        return carry

    lax.fori_loop(0, EPT // CH, chunk, 0)
    plsc.subcore_barrier()
    pltpu.sync_copy(acc_sh.at[pl.ds(s * RPT, RPT)],
                    sums_h.at[c, pl.ds(s * RPT, RPT)])
    pltpu.sync_copy(cnt_sh.at[pl.ds(s * RPT, RPT)],
                    cnts_h.at[c, pl.ds(s * RPT, RPT)])


def _aggregate(src_p, dst_p, x_embed):
    z2 = jnp.zeros((RPT, F), jnp.float32)
    z1 = jnp.zeros((RPT,), jnp.float32)
    run = pl.kernel(
        _sc_body,
        out_type=[
            jax.ShapeDtypeStruct((NC, NPAD, F), jnp.float32),
            jax.ShapeDtypeStruct((NC, NPAD), jnp.float32),
        ],
        mesh=plsc.VectorSubcoreMesh(core_axis_name="c", subcore_axis_name="s"),
        scratch_types=[
            pltpu.VMEM_SHARED((NPAD, F), jnp.float32),
            pltpu.VMEM_SHARED((NPAD,), jnp.float32),
            pltpu.VMEM((CH,), jnp.int32),
            pltpu.VMEM((CH,), jnp.int32),
            pltpu.VMEM((CH, F), jnp.float32),
            pltpu.VMEM((CH,), jnp.float32),
            pltpu.SemaphoreType.DMA,
        ],
    )
    return run(src_p, dst_p, x_embed, z2, z1)


def _fuse_bn_body(xe_ref, p_ref, cnt_ref, wf_ref, bf_ref, g_ref, bt_ref,
                  out_ref, fused_sc, s_sc, q_sc):
    ph = pl.program_id(0)
    i = pl.program_id(1)

    @pl.when(ph == 0)
    def _():
        summed = p_ref[0] + p_ref[1]
        cnt = cnt_ref[0] + cnt_ref[1]
        aggs = jnp.where(cnt > 0.0, summed / jnp.maximum(cnt, 1.0), 0.0)
        dn = (((1,), (1,)), ((), ()))
        fused = (
            lax.dot_general(xe_ref[...], wf_ref[...][:, :F], dn,
                            preferred_element_type=jnp.float32)
            + lax.dot_general(aggs, wf_ref[...][:, F:], dn,
                              preferred_element_type=jnp.float32)
            + bf_ref[...])
        fused_sc[pl.ds(i * RB, RB), :] = fused

        @pl.when(i == 0)
        def _():
            s_sc[...] = jnp.zeros_like(s_sc)
            q_sc[...] = jnp.zeros_like(q_sc)

        s_sc[...] += jnp.sum(fused, axis=0, keepdims=True)
        q_sc[...] += jnp.sum(fused * fused, axis=0, keepdims=True)

    @pl.when(ph == 1)
    def _():
        mean = s_sc[...] * (1.0 / N)
        var = q_sc[...] * (1.0 / N) - mean * mean
        inv = lax.rsqrt(var + 1e-5)
        xh = ((fused_sc[pl.ds(i * RB, RB), :] - mean) * inv * g_ref[...]
              + bt_ref[...])
        out_ref[...] = 0.5 * xh * (1.0 + lax.erf(xh * INV_SQRT2))


def _fuse_bn(x_embed, sums, cnts, W_fus, b_fus, gamma, beta):
    return pl.pallas_call(
        _fuse_bn_body,
        grid=(2, N // RB),
        in_specs=[
            pl.BlockSpec((RB, F), lambda p, i: (i * (1 - p), 0)),
            pl.BlockSpec((NC, RB, F), lambda p, i: (0, i * (1 - p), 0)),
            pl.BlockSpec((NC, RB, 1), lambda p, i: (0, i * (1 - p), 0)),
            pl.BlockSpec((F, 2 * F), lambda p, i: (0, 0)),
            pl.BlockSpec((1, F), lambda p, i: (0, 0)),
            pl.BlockSpec((1, F), lambda p, i: (0, 0)),
            pl.BlockSpec((1, F), lambda p, i: (0, 0)),
        ],
        out_specs=pl.BlockSpec((RB, F), lambda p, i: (i, 0)),
        out_shape=jax.ShapeDtypeStruct((N, F), jnp.float32),
        scratch_shapes=[
            pltpu.VMEM((N, F), jnp.float32),
            pltpu.VMEM((1, F), jnp.float32),
            pltpu.VMEM((1, F), jnp.float32),
        ],
    )(x_embed, sums, cnts.reshape(NC, NPAD, 1), W_fus, b_fus.reshape(1, F),
      gamma.reshape(1, F), beta.reshape(1, F))


def kernel(x, edge_index, B, W_proj, b_proj, W_fus, b_fus, gamma, beta):
    x_embed = _embed(x, B, W_proj, b_proj)
    src_p = jnp.concatenate(
        [edge_index[0], jnp.zeros((EPAD - E,), jnp.int32)])
    dst_p = jnp.concatenate(
        [edge_index[1], jnp.full((EPAD - E,), N, jnp.int32)])
    sums, cnts = _aggregate(src_p, dst_p, x_embed)
    return _fuse_bn(x_embed, sums, cnts, W_fus, b_fus, gamma, beta)


# speedup vs baseline: 1.4862x; 1.0789x over previous
import functools
import math

import jax
import jax.numpy as jnp
from jax import lax
from jax.experimental import pallas as pl
from jax.experimental.pallas import tpu as pltpu
from jax.experimental.pallas import tpu_sc as plsc

N = 10000
F = 128
E = 320000
NC, NS = 2, 16
NW = NC * NS
CH = 128
EPT = 10112
EPAD = EPT * NW
NPAD = 10240
RPT = NPAD // NS
RB = 1000
TWO_PI = 2.0 * math.pi
INV_SQRT2 = 1.0 / math.sqrt(2.0)


def _embed_body(x_ref, b_ref, wp_ref, bp_ref, out_ref):
    xp = TWO_PI * jnp.dot(x_ref[...], b_ref[...],
                          preferred_element_type=jnp.float32)
    dn = (((1,), (1,)), ((), ()))
    out_ref[...] = (
        lax.dot_general(jnp.cos(xp), wp_ref[...][:, :F], dn,
                        preferred_element_type=jnp.float32)
        + lax.dot_general(jnp.sin(xp), wp_ref[...][:, F:], dn,
                          preferred_element_type=jnp.float32)
        + bp_ref[...])


def _embed(x, B, W_proj, b_proj):
    return pl.pallas_call(
        _embed_body,
        grid=(N // RB,),
        in_specs=[
            pl.BlockSpec((RB, F), lambda i: (i, 0)),
            pl.BlockSpec((F, F), lambda i: (0, 0)),
            pl.BlockSpec((F, 2 * F), lambda i: (0, 0)),
            pl.BlockSpec((1, F), lambda i: (0, 0)),
        ],
        out_specs=pl.BlockSpec((RB, F), lambda i: (i, 0)),
        out_shape=jax.ShapeDtypeStruct((N, F), jnp.float32),
    )(x, B, W_proj, b_proj.reshape(1, F))


def _sc_body(src_h, dst_h, xe_h, z2_h, z1_h, sums_h, cnts_h,
             acc_sh, cnt_sh, sidx, didx, rows, ones, sem):
    c = lax.axis_index("c")
    s = lax.axis_index("s")
    wid = c * NS + s
    for j in range(CH // 16):
        ones[pl.ds(j * 16, 16)] = jnp.ones((16,), jnp.float32)
    pltpu.sync_copy(z2_h, acc_sh.at[pl.ds(s * RPT, RPT)])
    pltpu.sync_copy(z1_h, cnt_sh.at[pl.ds(s * RPT, RPT)])
    plsc.subcore_barrier()
    base = wid * EPT

    def chunk(k, carry):
        off = base + k * CH
        pltpu.sync_copy(src_h.at[pl.ds(off, CH)], sidx)
        pltpu.sync_copy(dst_h.at[pl.ds(off, CH)], didx)
        pltpu.async_copy(xe_h.at[sidx], rows, sem).wait()
        pltpu.sync_copy(rows, acc_sh.at[didx], add=True)
        pltpu.sync_copy(ones, cnt_sh.at[didx], add=True)
        return carry

    lax.fori_loop(0, EPT // CH, chunk, 0)
    plsc.subcore_barrier()
    pltpu.sync_copy(acc_sh.at[pl.ds(s * RPT, RPT)],
                    sums_h.at[c, pl.ds(s * RPT, RPT)])
    pltpu.sync_copy(cnt_sh.at[pl.ds(s * RPT, RPT)],
                    cnts_h.at[c, pl.ds(s * RPT, RPT)])


def _aggregate(src_p, dst_p, x_embed):
    z2 = jnp.zeros((RPT, F), jnp.float32)
    z1 = jnp.zeros((RPT,), jnp.float32)
    run = pl.kernel(
        _sc_body,
        out_type=[
            jax.ShapeDtypeStruct((NC, NPAD, F), jnp.float32),
            jax.ShapeDtypeStruct((NC, NPAD), jnp.float32),
        ],
        mesh=plsc.VectorSubcoreMesh(core_axis_name="c", subcore_axis_name="s"),
        scratch_types=[
            pltpu.VMEM_SHARED((NPAD, F), jnp.float32),
            pltpu.VMEM_SHARED((NPAD,), jnp.float32),
            pltpu.VMEM((CH,), jnp.int32),
            pltpu.VMEM((CH,), jnp.int32),
            pltpu.VMEM((CH, F), jnp.float32),
            pltpu.VMEM((CH,), jnp.float32),
            pltpu.SemaphoreType.DMA,
        ],
    )
    return run(src_p, dst_p, x_embed, z2, z1)


def _fuse_body(xe_ref, p_ref, cnt_ref, wf_ref, bf_ref,
               fused_ref, s_ref, q_ref):
    i = pl.program_id(0)
    summed = p_ref[0] + p_ref[1]
    cnt = cnt_ref[0] + cnt_ref[1]
    aggs = jnp.where(cnt > 0.0, summed / jnp.maximum(cnt, 1.0), 0.0)
    dn = (((1,), (1,)), ((), ()))
    fused = (
        lax.dot_general(xe_ref[...], wf_ref[...][:, :F], dn,
                        preferred_element_type=jnp.float32)
        + lax.dot_general(aggs, wf_ref[...][:, F:], dn,
                          preferred_element_type=jnp.float32)
        + bf_ref[...])
    fused_ref[...] = fused

    @pl.when(i == 0)
    def _():
        s_ref[...] = jnp.zeros_like(s_ref)
        q_ref[...] = jnp.zeros_like(q_ref)

    s_ref[...] += jnp.sum(fused, axis=0, keepdims=True)
    q_ref[...] += jnp.sum(fused * fused, axis=0, keepdims=True)


def _fuse(x_embed, sums, cnts, W_fus, b_fus):
    return pl.pallas_call(
        _fuse_body,
        grid=(N // RB,),
        in_specs=[
            pl.BlockSpec((RB, F), lambda i: (i, 0)),
            pl.BlockSpec((NC, RB, F), lambda i: (0, i, 0)),
            pl.BlockSpec((NC, RB, 1), lambda i: (0, i, 0)),
            pl.BlockSpec((F, 2 * F), lambda i: (0, 0)),
            pl.BlockSpec((1, F), lambda i: (0, 0)),
        ],
        out_specs=[
            pl.BlockSpec((RB, F), lambda i: (i, 0)),
            pl.BlockSpec((1, F), lambda i: (0, 0)),
            pl.BlockSpec((1, F), lambda i: (0, 0)),
        ],
        out_shape=[
            jax.ShapeDtypeStruct((N, F), jnp.float32),
            jax.ShapeDtypeStruct((1, F), jnp.float32),
            jax.ShapeDtypeStruct((1, F), jnp.float32),
        ],
    )(x_embed, sums, cnts.reshape(NC, NPAD, 1), W_fus, b_fus.reshape(1, F))


def _bn_body(fused_ref, s_ref, q_ref, g_ref, bt_ref, out_ref):
    mean = s_ref[...] * (1.0 / N)
    var = q_ref[...] * (1.0 / N) - mean * mean
    inv = lax.rsqrt(var + 1e-5)
    xh = (fused_ref[...] - mean) * inv * g_ref[...] + bt_ref[...]
    out_ref[...] = 0.5 * xh * (1.0 + lax.erf(xh * INV_SQRT2))


def _bn_gelu(fused, ssum, sq, gamma, beta):
    return pl.pallas_call(
        _bn_body,
        grid=(N // RB,),
        in_specs=[
            pl.BlockSpec((RB, F), lambda i: (i, 0)),
            pl.BlockSpec((1, F), lambda i: (0, 0)),
            pl.BlockSpec((1, F), lambda i: (0, 0)),
            pl.BlockSpec((1, F), lambda i: (0, 0)),
            pl.BlockSpec((1, F), lambda i: (0, 0)),
        ],
        out_specs=pl.BlockSpec((RB, F), lambda i: (i, 0)),
        out_shape=jax.ShapeDtypeStruct((N, F), jnp.float32),
    )(fused, ssum, sq, gamma.reshape(1, F), beta.reshape(1, F))


def kernel(x, edge_index, B, W_proj, b_proj, W_fus, b_fus, gamma, beta):
    x_embed = _embed(x, B, W_proj, b_proj)
    src_p = jnp.concatenate(
        [edge_index[0], jnp.zeros((EPAD - E,), jnp.int32)])
    dst_p = jnp.concatenate(
        [edge_index[1], jnp.full((EPAD - E,), N, jnp.int32)])
    sums, cnts = _aggregate(src_p, dst_p, x_embed)
    fused, ssum, sq = _fuse(x_embed, sums, cnts, W_fus, b_fus)
    return _bn_gelu(fused, ssum, sq, gamma, beta)


# speedup vs baseline: 1.5918x; 1.0711x over previous
import functools
import math

import jax
import jax.numpy as jnp
from jax import lax
from jax.experimental import pallas as pl
from jax.experimental.pallas import tpu as pltpu
from jax.experimental.pallas import tpu_sc as plsc

N = 10000
F = 128
E = 320000
NC, NS = 2, 16
NW = NC * NS
CH = 128
EPT = 10112
EPAD = EPT * NW
NPAD = 10240
RPT = NPAD // NS
RB = 1000
TWO_PI = 2.0 * math.pi
INV_SQRT2 = 1.0 / math.sqrt(2.0)


def _embed_body(x_ref, b_ref, wp_ref, bp_ref, out_ref):
    xp = TWO_PI * jnp.dot(x_ref[...], b_ref[...],
                          preferred_element_type=jnp.float32)
    dn = (((1,), (1,)), ((), ()))
    out_ref[...] = (
        lax.dot_general(jnp.cos(xp), wp_ref[...][:, :F], dn,
                        preferred_element_type=jnp.float32)
        + lax.dot_general(jnp.sin(xp), wp_ref[...][:, F:], dn,
                          preferred_element_type=jnp.float32)
        + bp_ref[...])


def _embed(x, B, W_proj, b_proj):
    return pl.pallas_call(
        _embed_body,
        grid=(N // RB,),
        in_specs=[
            pl.BlockSpec((RB, F), lambda i: (i, 0)),
            pl.BlockSpec((F, F), lambda i: (0, 0)),
            pl.BlockSpec((F, 2 * F), lambda i: (0, 0)),
            pl.BlockSpec((1, F), lambda i: (0, 0)),
        ],
        out_specs=pl.BlockSpec((RB, F), lambda i: (i, 0)),
        out_shape=jax.ShapeDtypeStruct((N, F), jnp.float32),
    )(x, B, W_proj, b_proj.reshape(1, F))


def _sc_body(src_h, dst_h, xe_h, z2_h, z1_h, sums_h, cnts_h,
             acc_sh, cnt_sh, sidx, didx, rows, ones, sem):
    c = lax.axis_index("c")
    s = lax.axis_index("s")
    wid = c * NS + s
    for j in range(CH // 16):
        ones[pl.ds(j * 16, 16)] = jnp.ones((16,), jnp.float32)
    pltpu.sync_copy(z2_h, acc_sh.at[pl.ds(s * RPT, RPT)])
    pltpu.sync_copy(z1_h, cnt_sh.at[pl.ds(s * RPT, RPT)])
    plsc.subcore_barrier()
    base = wid * EPT

    def chunk(k, carry):
        off = base + k * CH
        pltpu.sync_copy(src_h.at[pl.ds(off, CH)], sidx)
        d = pltpu.async_copy(xe_h.at[sidx], rows, sem)
        pltpu.sync_copy(dst_h.at[pl.ds(off, CH)], didx)
        d.wait()
        pltpu.sync_copy(rows, acc_sh.at[didx], add=True)
        pltpu.sync_copy(ones, cnt_sh.at[didx], add=True)
        return carry

    lax.fori_loop(0, EPT // CH, chunk, 0)
    plsc.subcore_barrier()
    pltpu.sync_copy(acc_sh.at[pl.ds(s * RPT, RPT)],
                    sums_h.at[c, pl.ds(s * RPT, RPT)])
    pltpu.sync_copy(cnt_sh.at[pl.ds(s * RPT, RPT)],
                    cnts_h.at[c, pl.ds(s * RPT, RPT)])


def _aggregate(src_p, dst_p, x_embed):
    z2 = jnp.zeros((RPT, F), jnp.float32)
    z1 = jnp.zeros((RPT,), jnp.float32)
    run = pl.kernel(
        _sc_body,
        out_type=[
            jax.ShapeDtypeStruct((NC, NPAD, F), jnp.float32),
            jax.ShapeDtypeStruct((NC, NPAD), jnp.float32),
        ],
        mesh=plsc.VectorSubcoreMesh(core_axis_name="c", subcore_axis_name="s"),
        scratch_types=[
            pltpu.VMEM_SHARED((NPAD, F), jnp.float32),
            pltpu.VMEM_SHARED((NPAD,), jnp.float32),
            pltpu.VMEM((CH,), jnp.int32),
            pltpu.VMEM((CH,), jnp.int32),
            pltpu.VMEM((CH, F), jnp.float32),
            pltpu.VMEM((CH,), jnp.float32),
            pltpu.SemaphoreType.DMA,
        ],
    )
    return run(src_p, dst_p, x_embed, z2, z1)


def _fuse_body(xe_ref, p_ref, cnt_ref, wf_ref, bf_ref,
               fused_ref, s_ref, q_ref):
    i = pl.program_id(0)
    summed = p_ref[0] + p_ref[1]
    cnt = cnt_ref[0] + cnt_ref[1]
    aggs = jnp.where(cnt > 0.0, summed / jnp.maximum(cnt, 1.0), 0.0)
    dn = (((1,), (1,)), ((), ()))
    fused = (
        lax.dot_general(xe_ref[...], wf_ref[...][:, :F], dn,
                        preferred_element_type=jnp.float32)
        + lax.dot_general(aggs, wf_ref[...][:, F:], dn,
                          preferred_element_type=jnp.float32)
        + bf_ref[...])
    fused_ref[...] = fused

    @pl.when(i == 0)
    def _():
        s_ref[...] = jnp.zeros_like(s_ref)
        q_ref[...] = jnp.zeros_like(q_ref)

    s_ref[...] += jnp.sum(fused, axis=0, keepdims=True)
    q_ref[...] += jnp.sum(fused * fused, axis=0, keepdims=True)


def _fuse(x_embed, sums, cnts, W_fus, b_fus):
    return pl.pallas_call(
        _fuse_body,
        grid=(N // RB,),
        in_specs=[
            pl.BlockSpec((RB, F), lambda i: (i, 0)),
            pl.BlockSpec((NC, RB, F), lambda i: (0, i, 0)),
            pl.BlockSpec((NC, RB, 1), lambda i: (0, i, 0)),
            pl.BlockSpec((F, 2 * F), lambda i: (0, 0)),
            pl.BlockSpec((1, F), lambda i: (0, 0)),
        ],
        out_specs=[
            pl.BlockSpec((RB, F), lambda i: (i, 0)),
            pl.BlockSpec((1, F), lambda i: (0, 0)),
            pl.BlockSpec((1, F), lambda i: (0, 0)),
        ],
        out_shape=[
            jax.ShapeDtypeStruct((N, F), jnp.float32),
            jax.ShapeDtypeStruct((1, F), jnp.float32),
            jax.ShapeDtypeStruct((1, F), jnp.float32),
        ],
    )(x_embed, sums, cnts.reshape(NC, NPAD, 1), W_fus, b_fus.reshape(1, F))


def _bn_body(fused_ref, s_ref, q_ref, g_ref, bt_ref, out_ref):
    mean = s_ref[...] * (1.0 / N)
    var = q_ref[...] * (1.0 / N) - mean * mean
    inv = lax.rsqrt(var + 1e-5)
    xh = (fused_ref[...] - mean) * inv * g_ref[...] + bt_ref[...]
    out_ref[...] = 0.5 * xh * (1.0 + lax.erf(xh * INV_SQRT2))


def _bn_gelu(fused, ssum, sq, gamma, beta):
    return pl.pallas_call(
        _bn_body,
        grid=(N // RB,),
        in_specs=[
            pl.BlockSpec((RB, F), lambda i: (i, 0)),
            pl.BlockSpec((1, F), lambda i: (0, 0)),
            pl.BlockSpec((1, F), lambda i: (0, 0)),
            pl.BlockSpec((1, F), lambda i: (0, 0)),
            pl.BlockSpec((1, F), lambda i: (0, 0)),
        ],
        out_specs=pl.BlockSpec((RB, F), lambda i: (i, 0)),
        out_shape=jax.ShapeDtypeStruct((N, F), jnp.float32),
    )(fused, ssum, sq, gamma.reshape(1, F), beta.reshape(1, F))


def kernel(x, edge_index, B, W_proj, b_proj, W_fus, b_fus, gamma, beta):
    x_embed = _embed(x, B, W_proj, b_proj)
    src_p = jnp.concatenate(
        [edge_index[0], jnp.zeros((EPAD - E,), jnp.int32)])
    dst_p = jnp.concatenate(
        [edge_index[1], jnp.full((EPAD - E,), N, jnp.int32)])
    sums, cnts = _aggregate(src_p, dst_p, x_embed)
    fused, ssum, sq = _fuse(x_embed, sums, cnts, W_fus, b_fus)
    return _bn_gelu(fused, ssum, sq, gamma, beta)


# speedup vs baseline: 1.6918x; 1.0628x over previous
import functools
import math

import jax
import jax.numpy as jnp
from jax import lax
from jax.experimental import pallas as pl
from jax.experimental.pallas import tpu as pltpu
from jax.experimental.pallas import tpu_sc as plsc

N = 10000
F = 128
E = 320000
NC, NS = 2, 16
NW = NC * NS
CH = 128
EPT = 10112
EPAD = EPT * NW
NPAD = 10240
RPT = NPAD // NS
RB = 1000
TWO_PI = 2.0 * math.pi
INV_SQRT2 = 1.0 / math.sqrt(2.0)


def _embed_body(x_ref, b_ref, wp_ref, bp_ref, out_ref):
    xp = TWO_PI * jnp.dot(x_ref[...], b_ref[...],
                          preferred_element_type=jnp.float32)
    dn = (((1,), (1,)), ((), ()))
    out_ref[...] = (
        lax.dot_general(jnp.cos(xp), wp_ref[...][:, :F], dn,
                        preferred_element_type=jnp.float32)
        + lax.dot_general(jnp.sin(xp), wp_ref[...][:, F:], dn,
                          preferred_element_type=jnp.float32)
        + bp_ref[...])


def _embed(x, B, W_proj, b_proj):
    return pl.pallas_call(
        _embed_body,
        grid=(N // RB,),
        in_specs=[
            pl.BlockSpec((RB, F), lambda i: (i, 0)),
            pl.BlockSpec((F, F), lambda i: (0, 0)),
            pl.BlockSpec((F, 2 * F), lambda i: (0, 0)),
            pl.BlockSpec((1, F), lambda i: (0, 0)),
        ],
        out_specs=pl.BlockSpec((RB, F), lambda i: (i, 0)),
        out_shape=jax.ShapeDtypeStruct((N, F), jnp.float32),
    )(x, B, W_proj, b_proj.reshape(1, F))


def _sc_body(epk_h, xe_h, z2_h, z1_h, sums_h, cnts_h,
             acc_sh, cnt_sh, eidx, rows, ones, sem):
    c = lax.axis_index("c")
    s = lax.axis_index("s")
    wid = c * NS + s
    for j in range(CH // 16):
        ones[pl.ds(j * 16, 16)] = jnp.ones((16,), jnp.float32)
    pltpu.sync_copy(z2_h, acc_sh.at[pl.ds(s * RPT, RPT)])
    pltpu.sync_copy(z1_h, cnt_sh.at[pl.ds(s * RPT, RPT)])
    plsc.subcore_barrier()
    base = wid * (EPT // CH)

    def chunk(k, carry):
        pltpu.sync_copy(epk_h.at[base + k], eidx)
        pltpu.async_copy(xe_h.at[eidx.at[0]], rows, sem).wait()
        pltpu.sync_copy(rows, acc_sh.at[eidx.at[1]], add=True)
        pltpu.sync_copy(ones, cnt_sh.at[eidx.at[1]], add=True)
        return carry

    lax.fori_loop(0, EPT // CH, chunk, 0)
    plsc.subcore_barrier()
    pltpu.sync_copy(acc_sh.at[pl.ds(s * RPT, RPT)],
                    sums_h.at[c, pl.ds(s * RPT, RPT)])
    pltpu.sync_copy(cnt_sh.at[pl.ds(s * RPT, RPT)],
                    cnts_h.at[c, pl.ds(s * RPT, RPT)])


def _aggregate(epk, x_embed):
    z2 = jnp.zeros((RPT, F), jnp.float32)
    z1 = jnp.zeros((RPT,), jnp.float32)
    run = pl.kernel(
        _sc_body,
        out_type=[
            jax.ShapeDtypeStruct((NC, NPAD, F), jnp.float32),
            jax.ShapeDtypeStruct((NC, NPAD), jnp.float32),
        ],
        mesh=plsc.VectorSubcoreMesh(core_axis_name="c", subcore_axis_name="s"),
        scratch_types=[
            pltpu.VMEM_SHARED((NPAD, F), jnp.float32),
            pltpu.VMEM_SHARED((NPAD,), jnp.float32),
            pltpu.VMEM((2, CH), jnp.int32),
            pltpu.VMEM((CH, F), jnp.float32),
            pltpu.VMEM((CH,), jnp.float32),
            pltpu.SemaphoreType.DMA,
        ],
    )
    return run(epk, x_embed, z2, z1)


def _fuse_body(xe_ref, p_ref, cnt_ref, wf_ref, bf_ref,
               fused_ref, s_ref, q_ref):
    i = pl.program_id(0)
    summed = p_ref[0] + p_ref[1]
    cnt = cnt_ref[0] + cnt_ref[1]
    aggs = jnp.where(cnt > 0.0, summed / jnp.maximum(cnt, 1.0), 0.0)
    dn = (((1,), (1,)), ((), ()))
    fused = (
        lax.dot_general(xe_ref[...], wf_ref[...][:, :F], dn,
                        preferred_element_type=jnp.float32)
        + lax.dot_general(aggs, wf_ref[...][:, F:], dn,
                          preferred_element_type=jnp.float32)
        + bf_ref[...])
    fused_ref[...] = fused

    @pl.when(i == 0)
    def _():
        s_ref[...] = jnp.zeros_like(s_ref)
        q_ref[...] = jnp.zeros_like(q_ref)

    s_ref[...] += jnp.sum(fused, axis=0, keepdims=True)
    q_ref[...] += jnp.sum(fused * fused, axis=0, keepdims=True)


def _fuse(x_embed, sums, cnts, W_fus, b_fus):
    return pl.pallas_call(
        _fuse_body,
        grid=(N // RB,),
        in_specs=[
            pl.BlockSpec((RB, F), lambda i: (i, 0)),
            pl.BlockSpec((NC, RB, F), lambda i: (0, i, 0)),
            pl.BlockSpec((NC, RB, 1), lambda i: (0, i, 0)),
            pl.BlockSpec((F, 2 * F), lambda i: (0, 0)),
            pl.BlockSpec((1, F), lambda i: (0, 0)),
        ],
        out_specs=[
            pl.BlockSpec((RB, F), lambda i: (i, 0)),
            pl.BlockSpec((1, F), lambda i: (0, 0)),
            pl.BlockSpec((1, F), lambda i: (0, 0)),
        ],
        out_shape=[
            jax.ShapeDtypeStruct((N, F), jnp.float32),
            jax.ShapeDtypeStruct((1, F), jnp.float32),
            jax.ShapeDtypeStruct((1, F), jnp.float32),
        ],
    )(x_embed, sums, cnts.reshape(NC, NPAD, 1), W_fus, b_fus.reshape(1, F))


def _bn_body(fused_ref, s_ref, q_ref, g_ref, bt_ref, out_ref):
    mean = s_ref[...] * (1.0 / N)
    var = q_ref[...] * (1.0 / N) - mean * mean
    inv = lax.rsqrt(var + 1e-5)
    xh = (fused_ref[...] - mean) * inv * g_ref[...] + bt_ref[...]
    out_ref[...] = 0.5 * xh * (1.0 + lax.erf(xh * INV_SQRT2))


def _bn_gelu(fused, ssum, sq, gamma, beta):
    return pl.pallas_call(
        _bn_body,
        grid=(N // RB,),
        in_specs=[
            pl.BlockSpec((RB, F), lambda i: (i, 0)),
            pl.BlockSpec((1, F), lambda i: (0, 0)),
            pl.BlockSpec((1, F), lambda i: (0, 0)),
            pl.BlockSpec((1, F), lambda i: (0, 0)),
            pl.BlockSpec((1, F), lambda i: (0, 0)),
        ],
        out_specs=pl.BlockSpec((RB, F), lambda i: (i, 0)),
        out_shape=jax.ShapeDtypeStruct((N, F), jnp.float32),
    )(fused, ssum, sq, gamma.reshape(1, F), beta.reshape(1, F))


def kernel(x, edge_index, B, W_proj, b_proj, W_fus, b_fus, gamma, beta):
    x_embed = _embed(x, B, W_proj, b_proj)
    pad = jnp.stack([jnp.zeros((EPAD - E,), jnp.int32),
                     jnp.full((EPAD - E,), N, jnp.int32)])
    epk = jnp.concatenate([edge_index, pad], axis=1).reshape(
        2, EPAD // CH, CH).transpose(1, 0, 2)
    sums, cnts = _aggregate(epk, x_embed)
    fused, ssum, sq = _fuse(x_embed, sums, cnts, W_fus, b_fus)
    return _bn_gelu(fused, ssum, sq, gamma, beta)
